# baseline (speedup 1.0000x reference)
import jax
import jax.numpy as jnp
from jax import lax
from jax.experimental import pallas as pl
from jax.experimental.pallas import tpu as pltpu

D_INT = 32
D_MODEL = 1024
N_TOK = 4096
O_TILE = 128

A_ROWS = D_MODEL * D_MODEL * D_INT // 128
A_BLK = 16384
B_ROWS = D_MODEL * D_INT // 128


def _ray_body(Vt_ref, Rw_ref, Rb_ref, Wq_ref, bq_ref):
    c_idx = lax.broadcasted_iota(jnp.int32, (128, 4), 0)
    j_idx = lax.broadcasted_iota(jnp.int32, (128, 4), 1)
    Q = jnp.where(c_idx // D_INT == j_idx, Vt_ref[...], 0.0)
    Wq_ref[...] = jax.lax.dot_general(
        Rw_ref[...], Q, (((1,), (0,)), ((), ())))

    @pl.when(pl.program_id(0) == 0)
    def _():
        bq_ref[...] = jax.lax.dot_general(
            Rb_ref[...], Q, (((1,), (0,)), ((), ())))


def _mm_body(x_ref, Wr_ref, W0_ref, br_ref, b0_ref, out_ref):
    o = pl.program_id(0)
    Wt = Wr_ref[...] + W0_ref[...]
    acc = jax.lax.dot_general(x_ref[...], Wt, (((1,), (1,)), ((), ())))
    sl = pl.ds(o * O_TILE, O_TILE)
    out_ref[...] = acc + br_ref[:, sl] + b0_ref[:, sl]


def kernel(x, V, W0, b0, R_w, R_b):
    Rw128 = R_w.reshape(A_ROWS, 128)
    Rb128 = R_b.reshape(B_ROWS, 128)
    V_t = jnp.tile(V.reshape(1, D_INT), (1, 4)).reshape(128, 1)

    Wq, bq = pl.pallas_call(
        _ray_body,
        grid=(A_ROWS // A_BLK,),
        in_specs=[
            pl.BlockSpec((128, 1), lambda r: (0, 0)),
            pl.BlockSpec((A_BLK, 128), lambda r: (r, 0)),
            pl.BlockSpec((B_ROWS, 128), lambda r: (0, 0)),
        ],
        out_specs=[
            pl.BlockSpec((A_BLK, 4), lambda r: (r, 0)),
            pl.BlockSpec((B_ROWS, 4), lambda r: (0, 0)),
        ],
        out_shape=[
            jax.ShapeDtypeStruct((A_ROWS, 4), jnp.float32),
            jax.ShapeDtypeStruct((B_ROWS, 4), jnp.float32),
        ],
        compiler_params=pltpu.CompilerParams(
            dimension_semantics=("arbitrary",),
        ),
    )(V_t, Rw128, Rb128)

    Wray = Wq.reshape(D_MODEL, D_MODEL)
    bray = bq.reshape(1, D_MODEL)
    b02 = b0.reshape(1, D_MODEL)

    return pl.pallas_call(
        _mm_body,
        grid=(D_MODEL // O_TILE,),
        in_specs=[
            pl.BlockSpec((N_TOK, D_MODEL), lambda o: (0, 0)),
            pl.BlockSpec((O_TILE, D_MODEL), lambda o: (o, 0)),
            pl.BlockSpec((O_TILE, D_MODEL), lambda o: (o, 0)),
            pl.BlockSpec((1, D_MODEL), lambda o: (0, 0)),
            pl.BlockSpec((1, D_MODEL), lambda o: (0, 0)),
        ],
        out_specs=pl.BlockSpec((N_TOK, O_TILE), lambda o: (0, o)),
        out_shape=jax.ShapeDtypeStruct((N_TOK, D_MODEL), jnp.float32),
        compiler_params=pltpu.CompilerParams(
            dimension_semantics=("arbitrary",),
        ),
    )(x, Wray, W0, bray, b02)
